# baseline (speedup 1.0000x reference)
import functools

import jax
import jax.numpy as jnp
from jax import lax
from jax.experimental import pallas as pl
from jax.experimental.pallas import tpu as pltpu
from jax.experimental.pallas import tpu_sc as plsc

NC, NS = 2, 16
NW = NC * NS
D = 32
CHUNK = 1600
VB = 6400
BC = 16384


def _pack_table(tT):
    V = tT.shape[1]
    n_blocks = pl.cdiv(V, VB)

    def body(in_ref, out_ref):
        x = in_ref[...]
        z = jnp.transpose(x, (1, 0))
        w = z.reshape(VB // 4, 4, D)
        out_ref[...] = jnp.concatenate([w[:, j, :] for j in range(4)], axis=1)

    return pl.pallas_call(
        body,
        grid=(n_blocks,),
        in_specs=[pl.BlockSpec((D, VB), lambda i: (0, i))],
        out_specs=pl.BlockSpec((VB * D // 128, 128), lambda i: (i, 0)),
        out_shape=jax.ShapeDtypeStruct((V * D // 128, 128), jnp.float32),
    )(tT)


def _unpack_out(out_sc, H, Bc):
    q = Bc // 4
    r128 = Bc * D // 128
    x3 = out_sc.reshape(H, r128, 128)

    def body(in_ref, out_ref):
        x = in_ref[0]
        t = jnp.transpose(x, (1, 0))
        u = t.reshape(4, D, q)
        out_ref[0] = jnp.concatenate([u[j] for j in range(4)], axis=1)

    return pl.pallas_call(
        body,
        grid=(H,),
        in_specs=[pl.BlockSpec((1, r128, 128), lambda i: (i, 0, 0))],
        out_specs=pl.BlockSpec((1, D, Bc), lambda i: (i, 0, 0)),
        out_shape=jax.ShapeDtypeStruct((H, D, Bc), jnp.float32),
    )(x3)


@functools.cache
def _emb_kernel(B):
    b_per_w = B // NW
    n_chunks = b_per_w // CHUNK
    n_pairs = n_chunks // 2
    mesh = plsc.VectorSubcoreMesh(core_axis_name="c", subcore_axis_name="s")

    @functools.partial(
        pl.kernel,
        out_type=jax.ShapeDtypeStruct((B, D), jnp.float32),
        mesh=mesh,
        scratch_types=[
            pltpu.VMEM((2, CHUNK), jnp.int32),
            pltpu.VMEM((2, CHUNK, D), jnp.float32),
            pltpu.SemaphoreType.DMA,
            pltpu.SemaphoreType.DMA,
            pltpu.SemaphoreType.DMA,
            pltpu.SemaphoreType.DMA,
        ],
        compiler_params=pltpu.CompilerParams(use_tc_tiling_on_sc=False),
    )
    def emb(idx_hbm, table_hbm, out_hbm, idx_v, rows_v, g0, g1, s0, s1):
        wid = lax.axis_index("s") * NC + lax.axis_index("c")
        base = wid * b_per_w
        last = n_chunks - 1

        def idx_load(buf, c):
            off = base + lax.min(c, last) * CHUNK
            pltpu.sync_copy(idx_hbm.at[pl.ds(off, CHUNK)], idx_v.at[buf])

        def gather_start(buf, sem):
            return pltpu.async_copy(table_hbm.at[idx_v.at[buf]], rows_v.at[buf], sem)

        def gather_wait(buf, sem):
            pltpu.make_async_copy(table_hbm.at[idx_v.at[buf]], rows_v.at[buf], sem).wait()

        def store_start(buf, c, sem):
            off = base + c * CHUNK
            return pltpu.async_copy(rows_v.at[buf], out_hbm.at[pl.ds(off, CHUNK)], sem)

        def store_wait(buf, sem):
            pltpu.make_async_copy(
                rows_v.at[buf], out_hbm.at[pl.ds(base, CHUNK)], sem
            ).wait()

        idx_load(0, 0)
        gather_start(0, g0)
        idx_load(1, 1)

        def body(p, carry):
            c0 = 2 * p
            c1 = c0 + 1
            gather_wait(0, g0)
            store_start(0, c0, s0)

            @pl.when(p > 0)
            def _():
                store_wait(1, s1)

            gather_start(1, g1)
            idx_load(0, c0 + 2)
            gather_wait(1, g1)
            store_start(1, c1, s1)
            store_wait(0, s0)

            @pl.when(p < n_pairs - 1)
            def _():
                gather_start(0, g0)

            idx_load(1, c1 + 2)
            return carry

        lax.fori_loop(0, n_pairs, body, 0)
        store_wait(1, s1)

    return emb


def kernel(tensor, table):
    nb, nh = tensor.shape
    B = nb * nh
    V = table.shape[0]
    r = jnp.arange(B, dtype=jnp.int32)
    perm = (r & ~jnp.int32(BC - 1)) | ((r & 3) * (BC // 4)) | ((r & (BC - 1)) >> 2)
    idx = jnp.take(tensor.T.astype(jnp.int32).reshape(B), perm)
    packed = _pack_table(table.T)
    table_lin = packed.reshape(V, D)
    G = 4
    hg = nh // G
    Bg = B // G
    parts = []
    for g in range(G):
        out_g = _emb_kernel(Bg)(
            lax.dynamic_slice_in_dim(idx, g * Bg, Bg), table_lin
        )
        parts.append(_unpack_out(out_g, hg, nb))
    out_hdb = jnp.concatenate(parts, axis=0)
    return jnp.transpose(out_hdb, (2, 0, 1))


# speedup vs baseline: 1.2027x; 1.2027x over previous
import functools

import jax
import jax.numpy as jnp
from jax import lax
from jax.experimental import pallas as pl
from jax.experimental.pallas import tpu as pltpu
from jax.experimental.pallas import tpu_sc as plsc

NC, NS = 2, 16
NW = NC * NS
D = 32
CHUNK = 1600
VB = 6400
BC = 16384


def _pack_table(tT):
    V = tT.shape[1]
    n_blocks = pl.cdiv(V, VB)

    def body(in_ref, out_ref):
        x = in_ref[...]
        z = jnp.transpose(x, (1, 0))
        w = z.reshape(VB // 4, 4, D)
        out_ref[...] = jnp.concatenate([w[:, j, :] for j in range(4)], axis=1)

    return pl.pallas_call(
        body,
        grid=(n_blocks,),
        in_specs=[pl.BlockSpec((D, VB), lambda i: (0, i))],
        out_specs=pl.BlockSpec((VB * D // 128, 128), lambda i: (i, 0)),
        out_shape=jax.ShapeDtypeStruct((V * D // 128, 128), jnp.float32),
    )(tT)


def _unpack_out(out_sc, H, Bc):
    q = Bc // 4
    r128 = Bc * D // 128
    x3 = out_sc.reshape(H, r128, 128)

    def body(in_ref, out_ref):
        x = in_ref[0]
        t = jnp.transpose(x, (1, 0))
        u = t.reshape(4, D, q)
        out_ref[0] = jnp.concatenate([u[j] for j in range(4)], axis=1)

    return pl.pallas_call(
        body,
        grid=(H,),
        in_specs=[pl.BlockSpec((1, r128, 128), lambda i: (i, 0, 0))],
        out_specs=pl.BlockSpec((1, D, Bc), lambda i: (i, 0, 0)),
        out_shape=jax.ShapeDtypeStruct((H, D, Bc), jnp.float32),
    )(x3)


@functools.cache
def _emb_kernel(B):
    b_per_w = B // NW
    n_chunks = b_per_w // CHUNK
    n_pairs = n_chunks // 2
    mesh = plsc.VectorSubcoreMesh(core_axis_name="c", subcore_axis_name="s")

    @functools.partial(
        pl.kernel,
        out_type=jax.ShapeDtypeStruct((B, D), jnp.float32),
        mesh=mesh,
        scratch_types=[
            pltpu.VMEM((2, CHUNK), jnp.int32),
            pltpu.VMEM((2, CHUNK, D), jnp.float32),
            pltpu.SemaphoreType.DMA,
            pltpu.SemaphoreType.DMA,
            pltpu.SemaphoreType.DMA,
            pltpu.SemaphoreType.DMA,
        ],
        compiler_params=pltpu.CompilerParams(use_tc_tiling_on_sc=False),
    )
    def emb(idx_hbm, table_hbm, out_hbm, idx_v, rows_v, g0, g1, s0, s1):
        wid = lax.axis_index("s") * NC + lax.axis_index("c")
        base = wid * b_per_w
        last = n_chunks - 1

        def idx_load(buf, c):
            off = base + lax.min(c, last) * CHUNK
            pltpu.sync_copy(idx_hbm.at[pl.ds(off, CHUNK)], idx_v.at[buf])

        def gather_start(buf, sem):
            return pltpu.async_copy(table_hbm.at[idx_v.at[buf]], rows_v.at[buf], sem)

        def gather_wait(buf, sem):
            pltpu.make_async_copy(table_hbm.at[idx_v.at[buf]], rows_v.at[buf], sem).wait()

        def store_start(buf, c, sem):
            off = base + c * CHUNK
            return pltpu.async_copy(rows_v.at[buf], out_hbm.at[pl.ds(off, CHUNK)], sem)

        def store_wait(buf, sem):
            pltpu.make_async_copy(
                rows_v.at[buf], out_hbm.at[pl.ds(base, CHUNK)], sem
            ).wait()

        idx_load(0, 0)
        gather_start(0, g0)
        idx_load(1, 1)

        def body(p, carry):
            c0 = 2 * p
            c1 = c0 + 1
            gather_wait(0, g0)
            store_start(0, c0, s0)

            @pl.when(p > 0)
            def _():
                store_wait(1, s1)

            gather_start(1, g1)
            idx_load(0, c0 + 2)
            gather_wait(1, g1)
            store_start(1, c1, s1)
            store_wait(0, s0)

            @pl.when(p < n_pairs - 1)
            def _():
                gather_start(0, g0)

            idx_load(1, c1 + 2)
            return carry

        lax.fori_loop(0, n_pairs, body, 0)
        store_wait(1, s1)

    return emb


def kernel(tensor, table):
    nb, nh = tensor.shape
    B = nb * nh
    V = table.shape[0]
    r = jnp.arange(B, dtype=jnp.int32)
    perm = (r & ~jnp.int32(BC - 1)) | ((r & 3) * (BC // 4)) | ((r & (BC - 1)) >> 2)
    idx = jnp.take(tensor.T.astype(jnp.int32).reshape(B), perm)
    packed = _pack_table(table.T)
    table_lin = packed.reshape(V, D)
    out_sc = _emb_kernel(B)(idx, table_lin)
    out_hdb = _unpack_out(out_sc, nh, nb)
    return jnp.transpose(out_hdb, (2, 0, 1))
